# baseline (speedup 1.0000x reference)
import functools

import jax
import jax.numpy as jnp
from jax.experimental import pallas as pl
import jax.experimental.pallas.tpu as pltpu


def _blk_kernel(idx_ref, w_ref, x_ref, b_ref, out_ref, *, in_features, bo):
    fan = idx_ref.shape[0]
    iota = jax.lax.broadcasted_iota(jnp.int32, (in_features, bo), 0)
    idx = idx_ref[...]
    w = w_ref[...]
    s = jnp.zeros((in_features, bo), jnp.float32)
    for f in range(fan):
        s = s + jnp.where(iota == idx[f : f + 1, :], w[f : f + 1, :], 0.0)
    out_ref[...] = (
        jnp.dot(
            x_ref[...],
            s.astype(jnp.bfloat16),
            preferred_element_type=jnp.float32,
        )
        + b_ref[...]
    )


def kernel(input, weight, bias, indx_seqs):
    batch, in_features = input.shape
    out_features, fan_in = weight.shape
    bo = min(256, out_features)
    n_blk = out_features // bo

    idx_t = indx_seqs.astype(jnp.int32).T
    w_t = weight.T
    bias2 = bias.reshape(1, out_features)
    x_bf16 = input.astype(jnp.bfloat16)

    grid = (n_blk,)
    out = pl.pallas_call(
        functools.partial(_blk_kernel, in_features=in_features, bo=bo),
        grid=grid,
        in_specs=[
            pl.BlockSpec((fan_in, bo), lambda j: (0, j)),
            pl.BlockSpec((fan_in, bo), lambda j: (0, j)),
            pl.BlockSpec((batch, in_features), lambda j: (0, 0)),
            pl.BlockSpec((1, bo), lambda j: (0, j)),
        ],
        out_specs=pl.BlockSpec((batch, bo), lambda j: (0, j)),
        out_shape=jax.ShapeDtypeStruct((batch, out_features), jnp.float32),
    )(idx_t, w_t, x_bf16, bias2)
    return out


# speedup vs baseline: 1.4326x; 1.4326x over previous
import functools

import jax
import jax.numpy as jnp
from jax import lax
from jax.experimental import pallas as pl
import jax.experimental.pallas.tpu as pltpu
from jax.experimental.pallas import tpu_sc as plsc


_NBUF = 4


def _sc_densify_body(idx_hbm, w_hbm, s_hbm, idx_v, w_v, sems, *rbufs,
                     out_features, in_features, fan_in, n_workers):
    nc = plsc.get_sparse_core_info().num_cores
    wid = lax.axis_index("s") * nc + lax.axis_index("c")
    rows = out_features // n_workers
    base = wid * rows
    elems = rows * fan_in

    pltpu.sync_copy(idx_hbm.at[pl.ds(base * fan_in, elems)], idx_v)
    pltpu.sync_copy(w_hbm.at[pl.ds(base * fan_in, elems)], w_v)

    zeros16 = jnp.zeros((16,), jnp.float32)

    def zero_body(i, _):
        for b in range(_NBUF):
            rbufs[b][pl.ds(i * 16, 16)] = zeros16
        return 0

    lax.fori_loop(0, in_features // 16, zero_body, 0)

    nvec = fan_in // 16
    copies = [None] * _NBUF
    for r in range(rows):
        b = r % _NBUF
        if copies[b] is not None:
            copies[b].wait()
            prev = r - _NBUF
            for h in range(nvec):
                iv = idx_v[pl.ds(prev * fan_in + h * 16, 16)]
                plsc.store_scatter(rbufs[b], [iv], zeros16)
        for h in range(nvec):
            iv = idx_v[pl.ds(r * fan_in + h * 16, 16)]
            wv = w_v[pl.ds(r * fan_in + h * 16, 16)]
            plsc.store_scatter(rbufs[b], [iv], wv)
        copies[b] = pltpu.async_copy(rbufs[b], s_hbm.at[base + r], sems[b])
    for b in range(_NBUF):
        if copies[b] is not None:
            copies[b].wait()


def _sc_densify(idx_flat, w_flat, out_features, in_features, fan_in):
    info = plsc.get_sparse_core_info()
    n_workers = info.num_cores * info.num_subcores
    mesh = plsc.VectorSubcoreMesh(core_axis_name="c", subcore_axis_name="s")
    elems = (out_features // n_workers) * fan_in
    kern = pl.kernel(
        functools.partial(
            _sc_densify_body,
            out_features=out_features,
            in_features=in_features,
            fan_in=fan_in,
            n_workers=n_workers,
        ),
        out_type=jax.ShapeDtypeStruct((out_features, in_features), jnp.float32),
        mesh=mesh,
        scratch_types=[
            pltpu.VMEM((elems,), jnp.int32),
            pltpu.VMEM((elems,), jnp.float32),
            [pltpu.SemaphoreType.DMA] * _NBUF,
        ]
        + [pltpu.VMEM((in_features,), jnp.float32)] * _NBUF,
        compiler_params=pltpu.CompilerParams(needs_layout_passes=False),
    )
    return kern(idx_flat, w_flat)


def _mm_kernel(x_ref, s_ref, b_ref, out_ref):
    out_ref[...] = (
        lax.dot_general(
            x_ref[...],
            s_ref[...],
            (((1,), (1,)), ((), ())),
            preferred_element_type=jnp.float32,
        )
        + b_ref[...]
    )


def kernel(input, weight, bias, indx_seqs):
    batch, in_features = input.shape
    out_features, fan_in = weight.shape
    idx_flat = indx_seqs.astype(jnp.int32).reshape(-1)
    w_flat = weight.reshape(-1)

    s = _sc_densify(idx_flat, w_flat, out_features, in_features, fan_in)

    bo = min(256, out_features)
    n_blk = out_features // bo
    bias2 = bias.reshape(1, out_features)
    out = pl.pallas_call(
        _mm_kernel,
        grid=(n_blk,),
        in_specs=[
            pl.BlockSpec((batch, in_features), lambda j: (0, 0)),
            pl.BlockSpec((bo, in_features), lambda j: (j, 0)),
            pl.BlockSpec((1, bo), lambda j: (0, j)),
        ],
        out_specs=pl.BlockSpec((batch, bo), lambda j: (0, j)),
        out_shape=jax.ShapeDtypeStruct((batch, out_features), jnp.float32),
    )(input, s, bias2)
    return out


# speedup vs baseline: 1.8521x; 1.2928x over previous
import functools

import jax
import jax.numpy as jnp
from jax.experimental import pallas as pl


def _blk_kernel(idx_ref, w_ref, x_ref, b_ref, out_ref, *, in_features, bo):
    fan = idx_ref.shape[0]
    iota = jax.lax.broadcasted_iota(jnp.int16, (in_features, bo), 0)
    idx = idx_ref[...]
    w = w_ref[...]
    s = jnp.zeros((in_features, bo), jnp.bfloat16)
    for f in range(fan):
        s = jnp.where(iota == idx[f : f + 1, :], w[f : f + 1, :], s)
    out_ref[...] = (
        jnp.dot(
            x_ref[...],
            s.astype(jnp.float32),
            preferred_element_type=jnp.float32,
        )
        + b_ref[...]
    )


def kernel(input, weight, bias, indx_seqs):
    batch, in_features = input.shape
    out_features, fan_in = weight.shape
    bo = min(256, out_features)
    n_blk = out_features // bo

    idx_t = indx_seqs.astype(jnp.int16).T
    w_t = weight.T.astype(jnp.bfloat16)
    bias2 = bias.reshape(1, out_features)

    out = pl.pallas_call(
        functools.partial(_blk_kernel, in_features=in_features, bo=bo),
        grid=(n_blk,),
        in_specs=[
            pl.BlockSpec((fan_in, bo), lambda j: (0, j)),
            pl.BlockSpec((fan_in, bo), lambda j: (0, j)),
            pl.BlockSpec((batch, in_features), lambda j: (0, 0)),
            pl.BlockSpec((1, bo), lambda j: (0, j)),
        ],
        out_specs=pl.BlockSpec((batch, bo), lambda j: (0, j)),
        out_shape=jax.ShapeDtypeStruct((batch, out_features), jnp.float32),
    )(idx_t, w_t, input, bias2)
    return out
